# baseline (speedup 1.0000x reference)
import functools

import jax
import jax.numpy as jnp
from jax import lax
from jax.experimental import pallas as pl
from jax.experimental.pallas import tpu as pltpu
from jax.experimental.pallas import tpu_sc as plsc

S, M, E, TOPK = 8192, 2048, 16, 2
CAPACITY = TOPK * (S // E)

NBG = 4
BTG = S // NBG
NT = 64
CHT = S // NT
LANES = 16
SUB = CHT // LANES
NBS = 8
BTS = S // NBS


def _gate_body(x_ref, w_ref, i0_ref, i1_ref, v0_ref, v1_ref,
               c0_ref, c1_ref, gsum_ref):
    xb = x_ref[...]
    w = w_ref[...]
    logits = lax.dot_general(xb, w, (((1,), (1,)), ((), ())),
                             preferred_element_type=jnp.float32)
    mx = jnp.max(logits, axis=1, keepdims=True)
    ex = jnp.exp(logits - mx)
    g = ex / jnp.sum(ex, axis=1, keepdims=True)
    iota = lax.broadcasted_iota(jnp.int32, (BTG, E), 1)
    v0 = jnp.max(g, axis=1, keepdims=True)
    i0 = jnp.min(jnp.where(g >= v0, iota, E), axis=1, keepdims=True)
    g1 = jnp.where(iota == i0, -jnp.inf, g)
    v1 = jnp.max(g1, axis=1, keepdims=True)
    i1 = jnp.min(jnp.where(g1 >= v1, iota, E), axis=1, keepdims=True)
    sub_io = lax.broadcasted_iota(jnp.int32, (BTG, 128), 0)
    lane_io = lax.broadcasted_iota(jnp.int32, (BTG, 128), 1)
    pmask = (lane_io == sub_io % 128).astype(jnp.float32)
    rows = BTG // 128
    sel = (lax.broadcasted_iota(jnp.int32, (rows, BTG), 1) // 128
           == lax.broadcasted_iota(jnp.int32, (rows, BTG), 0)
           ).astype(jnp.float32)

    def pack1(col_f):
        return lax.dot_general(sel, col_f * pmask, (((1,), (0,)), ((), ())),
                               preferred_element_type=jnp.float32)

    def pack(col):
        col_f = col.astype(jnp.float32)
        hi = col_f.astype(jnp.bfloat16).astype(jnp.float32)
        return pack1(hi) + pack1(col_f - hi)

    i0_ref[...] = pack1(i0.astype(jnp.float32)).astype(jnp.int32)
    i1_ref[...] = pack1(i1.astype(jnp.float32)).astype(jnp.int32)
    v0_ref[...] = pack(v0)
    v1_ref[...] = pack(v1)
    m0 = (iota == i0).astype(jnp.int32)
    m1 = (iota == i1).astype(jnp.int32)
    nch = BTG // CHT
    c0_ref[...] = jnp.sum(m0.reshape(nch, CHT, E), axis=1).reshape(nch, 1, E)
    c1_ref[...] = jnp.sum(m1.reshape(nch, CHT, E), axis=1).reshape(nch, 1, E)
    gsum_ref[...] = jnp.sum(g, axis=0).reshape(1, 1, E)


def _gate(x, wg):
    tok_spec = lambda dt: jax.ShapeDtypeStruct((S // 128, 128), dt)
    blk_spec = jax.ShapeDtypeStruct((NT, 1, E), jnp.int32)
    nch = BTG // CHT
    return pl.pallas_call(
        _gate_body,
        grid=(NBG,),
        in_specs=[
            pl.BlockSpec((BTG, M), lambda b: (b, 0)),
            pl.BlockSpec((E, M), lambda b: (0, 0)),
        ],
        out_specs=[
            pl.BlockSpec((BTG // 128, 128), lambda b: (b, 0)),
            pl.BlockSpec((BTG // 128, 128), lambda b: (b, 0)),
            pl.BlockSpec((BTG // 128, 128), lambda b: (b, 0)),
            pl.BlockSpec((BTG // 128, 128), lambda b: (b, 0)),
            pl.BlockSpec((nch, 1, E), lambda b: (b, 0, 0)),
            pl.BlockSpec((nch, 1, E), lambda b: (b, 0, 0)),
            pl.BlockSpec((1, 1, E), lambda b: (b, 0, 0)),
        ],
        out_shape=[
            tok_spec(jnp.int32), tok_spec(jnp.int32),
            tok_spec(jnp.float32), tok_spec(jnp.float32),
            blk_spec, blk_spec,
            jax.ShapeDtypeStruct((NBG, 1, E), jnp.float32),
        ],
    )(x, wg)


def _make_route_body(half, with_laux):
    def _route_body(i0h, i1h, v0h, v1h, c0h, c1h, gsh, *out_and_scratch):
        if with_laux:
            (scale_h, laux_h,
             idx0, idx1, vv0, vv1, rank0, rank1, cnt0, cnt1,
             base0, base1, call0, call1, gb0r, gb1r, gt0r,
             sc_out, gs_l, tmp16,
             sem0, sem1, sem2, sem3, sem4, sem5) = out_and_scratch
        else:
            (scale_h,
             idx0, idx1, vv0, vv1, rank0, rank1, cnt0, cnt1,
             base0, base1, call0, call1, gb0r, gb1r, gt0r,
             sc_out, gs_l, tmp16,
             sem0, sem1, sem2, sem3, sem4, sem5) = out_and_scratch
        cid = lax.axis_index("c")
        sid = lax.axis_index("s")
        w = cid * 16 + sid
        wg_ = half * 32 + w
        base = wg_ * CHT
        d_i0 = pltpu.async_copy(i0h.at[pl.ds(base, CHT)], idx0, sem0)
        d_i1 = pltpu.async_copy(i1h.at[pl.ds(base, CHT)], idx1, sem1)
        d_v0 = pltpu.async_copy(v0h.at[pl.ds(base, CHT)], vv0, sem2)
        d_v1 = pltpu.async_copy(v1h.at[pl.ds(base, CHT)], vv1, sem3)
        d_c0 = pltpu.async_copy(c0h, call0, sem4)
        d_c1 = pltpu.async_copy(c1h, call1, sem5)

        zi = jnp.zeros((LANES,), jnp.int32)
        d_c0.wait()
        d_c1.wait()
        gb0 = zi
        gb1 = zi
        gt0 = zi
        for t in range(NT):
            r0 = call0[pl.ds(t * E, E)]
            r1 = call1[pl.ds(t * E, E)]
            mlt = jnp.where(t < wg_, 1, 0).astype(jnp.int32)
            gb0 = gb0 + r0 * mlt
            gb1 = gb1 + r1 * mlt
            gt0 = gt0 + r0
        gb0r[...] = gb0
        gb1r[...] = gb1
        gt0r[...] = gt0

        for j in range(LANES):
            cnt0[pl.ds(j * E, E)] = zi
            cnt1[pl.ds(j * E, E)] = zi

        lanes = lax.iota(jnp.int32, LANES)
        lane_off = lanes * SUB
        lane_cnt = lanes * E
        d_i0.wait()
        d_i1.wait()
        for t in range(SUB):
            pos = lane_off + t
            ids0 = plsc.load_gather(idx0, [pos])
            a0 = lane_cnt + ids0
            c0 = plsc.load_gather(cnt0, [a0])
            plsc.store_scatter(rank0, [pos], c0)
            plsc.store_scatter(cnt0, [a0], c0 + 1)
            ids1 = plsc.load_gather(idx1, [pos])
            a1 = lane_cnt + ids1
            c1 = plsc.load_gather(cnt1, [a1])
            plsc.store_scatter(rank1, [pos], c1)
            plsc.store_scatter(cnt1, [a1], c1 + 1)

        run0 = zi
        run1 = zi
        for l in range(LANES):
            base0[pl.ds(l * E, E)] = run0
            base1[pl.ds(l * E, E)] = run1
            run0 = run0 + cnt0[pl.ds(l * E, E)]
            run1 = run1 + cnt1[pl.ds(l * E, E)]

        d_v0.wait()
        d_v1.wait()
        for t in range(SUB):
            pos = lane_off + t
            ids0 = plsc.load_gather(idx0, [pos])
            r0 = plsc.load_gather(rank0, [pos])
            b0 = plsc.load_gather(base0, [lane_cnt + ids0])
            g0 = plsc.load_gather(gb0r, [ids0])
            loc0 = r0 + b0 + g0
            ids1 = plsc.load_gather(idx1, [pos])
            r1 = plsc.load_gather(rank1, [pos])
            b1 = plsc.load_gather(base1, [lane_cnt + ids1])
            g1 = plsc.load_gather(gb1r, [ids1])
            o1 = plsc.load_gather(gt0r, [ids1])
            loc1 = r1 + b1 + g1 + o1
            s0 = plsc.load_gather(vv0, [pos])
            s1 = plsc.load_gather(vv1, [pos])
            sc = (jnp.where(loc0 < CAPACITY, s0, 0.0)
                  + jnp.where(loc1 < CAPACITY, s1, 0.0))
            plsc.store_scatter(sc_out, [pos], sc)
        pltpu.sync_copy(sc_out, scale_h.at[pl.ds(w * CHT, CHT)])

        if with_laux:
            @pl.when(w == 0)
            def _laux():
                pltpu.sync_copy(gsh, gs_l)
                me_sum = jnp.zeros((E,), jnp.float32)
                for r in range(NBG):
                    me_sum = me_sum + gs_l[pl.ds(r * E, E)]
                me = me_sum * (1.0 / S)
                ce = gt0r[...].astype(jnp.float32) * (1.0 / S)
                tot = jnp.sum(me * ce) * float(E)
                tmp16[...] = jnp.broadcast_to(tot, (E,))
                pltpu.sync_copy(tmp16, laux_h)

    return _route_body


@functools.cache
def _build_route(half, with_laux):
    if with_laux:
        out_type = (jax.ShapeDtypeStruct((S // 2,), jnp.float32),
                    jax.ShapeDtypeStruct((E,), jnp.float32))
    else:
        out_type = jax.ShapeDtypeStruct((S // 2,), jnp.float32)
    return pl.kernel(
        _make_route_body(half, with_laux),
        out_type=out_type,
        mesh=plsc.VectorSubcoreMesh(core_axis_name="c", subcore_axis_name="s",
                                    num_cores=2, num_subcores=16),
        compiler_params=pltpu.CompilerParams(needs_layout_passes=False),
        scratch_types=[
            pltpu.VMEM((CHT,), jnp.int32),
            pltpu.VMEM((CHT,), jnp.int32),
            pltpu.VMEM((CHT,), jnp.float32),
            pltpu.VMEM((CHT,), jnp.float32),
            pltpu.VMEM((CHT,), jnp.int32),
            pltpu.VMEM((CHT,), jnp.int32),
            pltpu.VMEM((LANES * E,), jnp.int32),
            pltpu.VMEM((LANES * E,), jnp.int32),
            pltpu.VMEM((LANES * E,), jnp.int32),
            pltpu.VMEM((LANES * E,), jnp.int32),
            pltpu.VMEM((NT * E,), jnp.int32),
            pltpu.VMEM((NT * E,), jnp.int32),
            pltpu.VMEM((E,), jnp.int32),
            pltpu.VMEM((E,), jnp.int32),
            pltpu.VMEM((E,), jnp.int32),
            pltpu.VMEM((CHT,), jnp.float32),
            pltpu.VMEM((NBG * E,), jnp.float32),
            pltpu.VMEM((E,), jnp.float32),
            pltpu.SemaphoreType.DMA,
            pltpu.SemaphoreType.DMA,
            pltpu.SemaphoreType.DMA,
            pltpu.SemaphoreType.DMA,
            pltpu.SemaphoreType.DMA,
            pltpu.SemaphoreType.DMA,
        ],
    )


def _unpack_scale(s_blk):
    rows = BTS // 128
    rep = (lax.broadcasted_iota(jnp.int32, (BTS, rows), 1)
           == lax.broadcasted_iota(jnp.int32, (BTS, rows), 0) // 128
           ).astype(jnp.float32)
    s_hi = s_blk.astype(jnp.bfloat16).astype(jnp.float32)
    y = (lax.dot_general(rep, s_hi, (((1,), (0,)), ((), ())),
                         preferred_element_type=jnp.float32)
         + lax.dot_general(rep, s_blk - s_hi, (((1,), (0,)), ((), ())),
                           preferred_element_type=jnp.float32))
    row_io = lax.broadcasted_iota(jnp.int32, (BTS, 128), 0)
    lane_io = lax.broadcasted_iota(jnp.int32, (BTS, 128), 1)
    return jnp.sum(y * (lane_io == row_io % 128).astype(jnp.float32),
                   axis=1, keepdims=True)


def _scale_body_a(x_ref, s_ref, o_ref):
    o_ref[...] = x_ref[...] * _unpack_scale(s_ref[...])


def _scale_body_b(x_ref, s_ref, y_ref, o_ref):
    del y_ref
    o_ref[...] = x_ref[...] * _unpack_scale(s_ref[...])


def _scale_mul_a(x, scale_half0):
    return pl.pallas_call(
        _scale_body_a,
        grid=(NBS // 2,),
        in_specs=[
            pl.BlockSpec((BTS, M), lambda b: (b, 0)),
            pl.BlockSpec((BTS // 128, 128), lambda b: (b, 0)),
        ],
        out_specs=pl.BlockSpec((BTS, M), lambda b: (b, 0)),
        out_shape=jax.ShapeDtypeStruct((S, M), jnp.float32),
    )(x, scale_half0)


def _scale_mul_b(x, scale_half1, y_prev):
    return pl.pallas_call(
        _scale_body_b,
        grid=(NBS // 2,),
        in_specs=[
            pl.BlockSpec((BTS, M), lambda b: (b + NBS // 2, 0)),
            pl.BlockSpec((BTS // 128, 128), lambda b: (b, 0)),
            pl.BlockSpec(memory_space=pl.ANY),
        ],
        out_specs=pl.BlockSpec((BTS, M), lambda b: (b + NBS // 2, 0)),
        out_shape=jax.ShapeDtypeStruct((S, M), jnp.float32),
        input_output_aliases={2: 0},
    )(x, scale_half1, y_prev)


def kernel(x, wg_weight):
    i0, i1, v0, v1, c0, c1, gs = _gate(x, wg_weight)
    i0f = i0.reshape(S)
    i1f = i1.reshape(S)
    v0f = v0.reshape(S)
    v1f = v1.reshape(S)
    c0f = c0.reshape(NT * E)
    c1f = c1.reshape(NT * E)
    gsf = gs.reshape(NBG * E)
    sc0, laux = _build_route(0, True)(i0f, i1f, v0f, v1f, c0f, c1f, gsf)
    sc1 = _build_route(1, False)(i0f, i1f, v0f, v1f, c0f, c1f, gsf)
    y0 = _scale_mul_a(x, sc0.reshape(S // 256, 128))
    y = _scale_mul_b(x, sc1.reshape(S // 256, 128), y0)
    return y, laux[0]


# speedup vs baseline: 1.0752x; 1.0752x over previous
import functools

import jax
import jax.numpy as jnp
from jax import lax
from jax.experimental import pallas as pl
from jax.experimental.pallas import tpu as pltpu
from jax.experimental.pallas import tpu_sc as plsc

S, M, E, TOPK = 8192, 2048, 16, 2
CAPACITY = TOPK * (S // E)

NBG = 4
BTG = S // NBG
NT = 64
CHT = S // NT
LANES = 16
SUB = CHT // LANES
NBS = 8
BTS = S // NBS


def _gate_body(x_ref, w_ref, i0_ref, i1_ref, v0_ref, v1_ref,
               c0_ref, c1_ref, laux_ref, g_acc, c_acc):
    xb = x_ref[...]
    w = w_ref[...]
    logits = lax.dot_general(xb, w, (((1,), (1,)), ((), ())),
                             preferred_element_type=jnp.float32)
    mx = jnp.max(logits, axis=1, keepdims=True)
    ex = jnp.exp(logits - mx)
    g = ex / jnp.sum(ex, axis=1, keepdims=True)
    iota = lax.broadcasted_iota(jnp.int32, (BTG, E), 1)
    v0 = jnp.max(g, axis=1, keepdims=True)
    i0 = jnp.min(jnp.where(g >= v0, iota, E), axis=1, keepdims=True)
    g1 = jnp.where(iota == i0, -jnp.inf, g)
    v1 = jnp.max(g1, axis=1, keepdims=True)
    i1 = jnp.min(jnp.where(g1 >= v1, iota, E), axis=1, keepdims=True)
    sub_io = lax.broadcasted_iota(jnp.int32, (BTG, 128), 0)
    lane_io = lax.broadcasted_iota(jnp.int32, (BTG, 128), 1)
    pmask = (lane_io == sub_io % 128).astype(jnp.float32)
    rows = BTG // 128
    sel = (lax.broadcasted_iota(jnp.int32, (rows, BTG), 1) // 128
           == lax.broadcasted_iota(jnp.int32, (rows, BTG), 0)
           ).astype(jnp.float32)

    def pack1(col_f):
        return lax.dot_general(sel, col_f * pmask, (((1,), (0,)), ((), ())),
                               preferred_element_type=jnp.float32)

    def pack(col):
        col_f = col.astype(jnp.float32)
        hi = col_f.astype(jnp.bfloat16).astype(jnp.float32)
        return pack1(hi) + pack1(col_f - hi)

    i0_ref[...] = pack1(i0.astype(jnp.float32)).astype(jnp.int32)
    i1_ref[...] = pack1(i1.astype(jnp.float32)).astype(jnp.int32)
    v0_ref[...] = pack(v0)
    v1_ref[...] = pack(v1)
    m0 = (iota == i0).astype(jnp.float32)
    m1 = (iota == i1).astype(jnp.float32)
    nch = BTG // CHT
    c0b = jnp.sum(m0.reshape(nch, CHT, E), axis=1)
    c1b = jnp.sum(m1.reshape(nch, CHT, E), axis=1)
    rep16 = (lax.broadcasted_iota(jnp.int32, (E, 128), 1) % E
             == lax.broadcasted_iota(jnp.int32, (E, 128), 0)
             ).astype(jnp.float32)
    maskA = (lax.broadcasted_iota(jnp.int32, (nch, 128), 1) // E
             == lax.broadcasted_iota(jnp.int32, (nch, 128), 0) % 8
             ).astype(jnp.float32)
    rowsel = (lax.broadcasted_iota(jnp.int32, (2, nch), 1) // 8
              == lax.broadcasted_iota(jnp.int32, (2, nch), 0)
              ).astype(jnp.float32)

    def pack_counts(cb):
        c2 = lax.dot_general(cb, rep16, (((1,), (0,)), ((), ())),
                             preferred_element_type=jnp.float32)
        return lax.dot_general(rowsel, c2 * maskA, (((1,), (0,)), ((), ())),
                               preferred_element_type=jnp.float32)

    b = pl.program_id(0)
    c0_ref[pl.ds(b * 2, 2), :] = pack_counts(c0b).astype(jnp.int32)
    c1_ref[pl.ds(b * 2, 2), :] = pack_counts(c1b).astype(jnp.int32)
    g_acc[pl.ds(b, 1), :] = jnp.sum(g, axis=0).reshape(1, E)
    c_acc[pl.ds(b, 1), :] = jnp.sum(m0, axis=0).reshape(1, E)

    @pl.when(b == NBG - 1)
    def _laux():
        me = jnp.sum(g_acc[...], axis=0) * (1.0 / S)
        ce = jnp.sum(c_acc[...], axis=0) * (1.0 / S)
        tot = jnp.sum(me * ce) * float(E)
        laux_ref[...] = jnp.full((8, 128), tot, jnp.float32)


def _gate(x, wg):
    tok_spec = lambda dt: jax.ShapeDtypeStruct((S // 128, 128), dt)
    return pl.pallas_call(
        _gate_body,
        grid=(NBG,),
        in_specs=[
            pl.BlockSpec((BTG, M), lambda b: (b, 0)),
            pl.BlockSpec((E, M), lambda b: (0, 0)),
        ],
        out_specs=[
            pl.BlockSpec((BTG // 128, 128), lambda b: (b, 0)),
            pl.BlockSpec((BTG // 128, 128), lambda b: (b, 0)),
            pl.BlockSpec((BTG // 128, 128), lambda b: (b, 0)),
            pl.BlockSpec((BTG // 128, 128), lambda b: (b, 0)),
            pl.BlockSpec((NT // 8, 128), lambda b: (0, 0)),
            pl.BlockSpec((NT // 8, 128), lambda b: (0, 0)),
            pl.BlockSpec((8, 128), lambda b: (0, 0)),
        ],
        out_shape=[
            tok_spec(jnp.int32), tok_spec(jnp.int32),
            tok_spec(jnp.float32), tok_spec(jnp.float32),
            jax.ShapeDtypeStruct((NT // 8, 128), jnp.int32),
            jax.ShapeDtypeStruct((NT // 8, 128), jnp.int32),
            jax.ShapeDtypeStruct((8, 128), jnp.float32),
        ],
        scratch_shapes=[
            pltpu.VMEM((NBG, E), jnp.float32),
            pltpu.VMEM((NBG, E), jnp.float32),
        ],
    )(x, wg)


def _make_route_body(half):
    def _route_body(i0h, i1h, v0h, v1h, c0h, c1h, scale_h,
                    idx0, idx1, vv0, vv1, rank0, rank1, cnt0, cnt1,
                    base0, base1, call0, call1, gb0r, gb1r, gt0r,
                    sc_out, sem0, sem1, sem2, sem3, sem4, sem5):
        cid = lax.axis_index("c")
        sid = lax.axis_index("s")
        w = cid * 16 + sid
        wg_ = half * 32 + w
        base = wg_ * CHT
        d_i0 = pltpu.async_copy(i0h.at[pl.ds(base, CHT)], idx0, sem0)
        d_i1 = pltpu.async_copy(i1h.at[pl.ds(base, CHT)], idx1, sem1)
        d_v0 = pltpu.async_copy(v0h.at[pl.ds(base, CHT)], vv0, sem2)
        d_v1 = pltpu.async_copy(v1h.at[pl.ds(base, CHT)], vv1, sem3)
        d_c0 = pltpu.async_copy(c0h, call0, sem4)
        d_c1 = pltpu.async_copy(c1h, call1, sem5)

        zi = jnp.zeros((LANES,), jnp.int32)
        d_c0.wait()
        d_c1.wait()
        gb0 = zi
        gb1 = zi
        gt0 = zi
        for t in range(NT):
            r0 = call0[pl.ds(t * E, E)]
            r1 = call1[pl.ds(t * E, E)]
            mlt = jnp.where(t < wg_, 1, 0).astype(jnp.int32)
            gb0 = gb0 + r0 * mlt
            gb1 = gb1 + r1 * mlt
            gt0 = gt0 + r0
        gb0r[...] = gb0
        gb1r[...] = gb1
        gt0r[...] = gt0

        for j in range(LANES):
            cnt0[pl.ds(j * E, E)] = zi
            cnt1[pl.ds(j * E, E)] = zi

        lanes = lax.iota(jnp.int32, LANES)
        lane_off = lanes * SUB
        lane_cnt = lanes * E
        d_i0.wait()
        d_i1.wait()
        for t in range(SUB):
            pos = lane_off + t
            ids0 = plsc.load_gather(idx0, [pos])
            a0 = lane_cnt + ids0
            c0 = plsc.load_gather(cnt0, [a0])
            plsc.store_scatter(rank0, [pos], c0)
            plsc.store_scatter(cnt0, [a0], c0 + 1)
            ids1 = plsc.load_gather(idx1, [pos])
            a1 = lane_cnt + ids1
            c1 = plsc.load_gather(cnt1, [a1])
            plsc.store_scatter(rank1, [pos], c1)
            plsc.store_scatter(cnt1, [a1], c1 + 1)

        run0 = zi
        run1 = zi
        for l in range(LANES):
            base0[pl.ds(l * E, E)] = run0
            base1[pl.ds(l * E, E)] = run1
            run0 = run0 + cnt0[pl.ds(l * E, E)]
            run1 = run1 + cnt1[pl.ds(l * E, E)]

        d_v0.wait()
        d_v1.wait()
        for t in range(SUB):
            pos = lane_off + t
            ids0 = plsc.load_gather(idx0, [pos])
            r0 = plsc.load_gather(rank0, [pos])
            b0 = plsc.load_gather(base0, [lane_cnt + ids0])
            g0 = plsc.load_gather(gb0r, [ids0])
            loc0 = r0 + b0 + g0
            ids1 = plsc.load_gather(idx1, [pos])
            r1 = plsc.load_gather(rank1, [pos])
            b1 = plsc.load_gather(base1, [lane_cnt + ids1])
            g1 = plsc.load_gather(gb1r, [ids1])
            o1 = plsc.load_gather(gt0r, [ids1])
            loc1 = r1 + b1 + g1 + o1
            s0 = plsc.load_gather(vv0, [pos])
            s1 = plsc.load_gather(vv1, [pos])
            sc = (jnp.where(loc0 < CAPACITY, s0, 0.0)
                  + jnp.where(loc1 < CAPACITY, s1, 0.0))
            plsc.store_scatter(sc_out, [pos], sc)
        pltpu.sync_copy(sc_out, scale_h.at[pl.ds(w * CHT, CHT)])

    return _route_body


@functools.cache
def _build_route(half):
    return pl.kernel(
        _make_route_body(half),
        out_type=jax.ShapeDtypeStruct((S // 2,), jnp.float32),
        mesh=plsc.VectorSubcoreMesh(core_axis_name="c", subcore_axis_name="s",
                                    num_cores=2, num_subcores=16),
        compiler_params=pltpu.CompilerParams(needs_layout_passes=False),
        scratch_types=[
            pltpu.VMEM((CHT,), jnp.int32),
            pltpu.VMEM((CHT,), jnp.int32),
            pltpu.VMEM((CHT,), jnp.float32),
            pltpu.VMEM((CHT,), jnp.float32),
            pltpu.VMEM((CHT,), jnp.int32),
            pltpu.VMEM((CHT,), jnp.int32),
            pltpu.VMEM((LANES * E,), jnp.int32),
            pltpu.VMEM((LANES * E,), jnp.int32),
            pltpu.VMEM((LANES * E,), jnp.int32),
            pltpu.VMEM((LANES * E,), jnp.int32),
            pltpu.VMEM((NT * E,), jnp.int32),
            pltpu.VMEM((NT * E,), jnp.int32),
            pltpu.VMEM((E,), jnp.int32),
            pltpu.VMEM((E,), jnp.int32),
            pltpu.VMEM((E,), jnp.int32),
            pltpu.VMEM((CHT,), jnp.float32),
            pltpu.SemaphoreType.DMA,
            pltpu.SemaphoreType.DMA,
            pltpu.SemaphoreType.DMA,
            pltpu.SemaphoreType.DMA,
            pltpu.SemaphoreType.DMA,
            pltpu.SemaphoreType.DMA,
        ],
    )


def _unpack_scale(s_blk):
    rows = BTS // 128
    rep = (lax.broadcasted_iota(jnp.int32, (BTS, rows), 1)
           == lax.broadcasted_iota(jnp.int32, (BTS, rows), 0) // 128
           ).astype(jnp.float32)
    s_hi = s_blk.astype(jnp.bfloat16).astype(jnp.float32)
    y = (lax.dot_general(rep, s_hi, (((1,), (0,)), ((), ())),
                         preferred_element_type=jnp.float32)
         + lax.dot_general(rep, s_blk - s_hi, (((1,), (0,)), ((), ())),
                           preferred_element_type=jnp.float32))
    row_io = lax.broadcasted_iota(jnp.int32, (BTS, 128), 0)
    lane_io = lax.broadcasted_iota(jnp.int32, (BTS, 128), 1)
    return jnp.sum(y * (lane_io == row_io % 128).astype(jnp.float32),
                   axis=1, keepdims=True)


def _scale_body_a(x_ref, s_ref, o_ref):
    o_ref[...] = x_ref[...] * _unpack_scale(s_ref[...])


def _scale_body_b(x_ref, s_ref, y_ref, o_ref):
    del y_ref
    o_ref[...] = x_ref[...] * _unpack_scale(s_ref[...])


def _scale_mul_a(x, scale_half0):
    return pl.pallas_call(
        _scale_body_a,
        grid=(NBS // 2,),
        in_specs=[
            pl.BlockSpec((BTS, M), lambda b: (b, 0)),
            pl.BlockSpec((BTS // 128, 128), lambda b: (b, 0)),
        ],
        out_specs=pl.BlockSpec((BTS, M), lambda b: (b, 0)),
        out_shape=jax.ShapeDtypeStruct((S, M), jnp.float32),
    )(x, scale_half0)


def _scale_mul_b(x, scale_half1, y_prev):
    return pl.pallas_call(
        _scale_body_b,
        grid=(NBS // 2,),
        in_specs=[
            pl.BlockSpec((BTS, M), lambda b: (b + NBS // 2, 0)),
            pl.BlockSpec((BTS // 128, 128), lambda b: (b, 0)),
            pl.BlockSpec(memory_space=pl.ANY),
        ],
        out_specs=pl.BlockSpec((BTS, M), lambda b: (b + NBS // 2, 0)),
        out_shape=jax.ShapeDtypeStruct((S, M), jnp.float32),
        input_output_aliases={2: 0},
    )(x, scale_half1, y_prev)


def kernel(x, wg_weight):
    i0, i1, v0, v1, c0p, c1p, laux_p = _gate(x, wg_weight)
    i0f = i0.reshape(S)
    i1f = i1.reshape(S)
    v0f = v0.reshape(S)
    v1f = v1.reshape(S)
    c0f = c0p.reshape(NT * E)
    c1f = c1p.reshape(NT * E)
    sc0 = _build_route(0)(i0f, i1f, v0f, v1f, c0f, c1f)
    sc1 = _build_route(1)(i0f, i1f, v0f, v1f, c0f, c1f)
    y0 = _scale_mul_a(x, sc0.reshape(S // 256, 128))
    y = _scale_mul_b(x, sc1.reshape(S // 256, 128), y0)
    return y, laux_p[0, 0]
